# baseline (speedup 1.0000x reference)
import functools

import jax
import jax.numpy as jnp
import numpy as np
from jax import lax
from jax.experimental import pallas as pl
from jax.experimental.pallas import tpu as pltpu
from jax.experimental.pallas import tpu_sc as plsc

BEAM = 16
VOCAB = 1_000_000
SEQ_LEN = 20
HALF = VOCAB // 2
NC, NS, L = 2, 16, 16
NW = NC * NS
CHUNK = 50_000
NCHUNK = HALF // CHUNK
G = 25
GROUPS = CHUNK // (G * L)

_NEG_INF = np.float32(-np.inf)


def _shard_topk_body(lp_hbm, keys_out, cols_out, buf0, buf1, kv_v, cv_v,
                     sem0, sem1):
    cid = lax.axis_index("c")
    sid = lax.axis_index("s")
    wid = sid * NC + cid
    base_elem = wid * HALF
    col0 = (wid % 2) * HALF
    iota = lax.iota(jnp.int32, L)
    is_odd = (wid % 2) == 1

    def src_slice(c):
        return lp_hbm.at[pl.ds(base_elem + c * CHUNK, CHUNK)]

    def patch_penalty(buf):
        x = buf[pl.ds(CHUNK - L, L)]
        x = x + jnp.where(iota == L - 1, np.float32(-1000.0),
                          np.float32(0.0))
        buf[pl.ds(CHUNK - L, L)] = x

    def process(buf, ccol0, carry):
        def group_body(g, carry):
            Tk, Tv, thr = carry
            gbase = g * (G * L)
            vecs = [buf[pl.ds(gbase + j * L, L)] for j in range(G)]
            ms = list(vecs)
            while len(ms) > 1:
                nxt = [jnp.maximum(ms[i], ms[i + 1])
                       for i in range(0, len(ms) - 1, 2)]
                if len(ms) % 2:
                    nxt.append(ms[-1])
                ms = nxt
            gmax = jnp.max(ms[0])

            def rescan(carry):
                Tk, Tv, thr = carry
                for j, X in enumerate(vecs):
                    xm = jnp.max(X)

                    def mrg(a, X=X, j=j):
                        Tk, Tv, _ = a
                        idxv = (ccol0 + gbase + j * L) + iota
                        sk, sv = plsc.sort_key_val(X, idxv, descending=True)
                        nk = jnp.maximum(Tk, sk)
                        nv = jnp.where(Tk >= sk, Tv, sv)
                        Tk2, Tv2 = plsc.sort_key_val(nk, nv)
                        return Tk2, Tv2, jnp.min(Tk2)

                    Tk, Tv, thr = lax.cond(xm > thr, mrg, lambda a: a,
                                           (Tk, Tv, thr))
                return Tk, Tv, thr

            return lax.cond(gmax > thr, rescan, lambda a: a, (Tk, Tv, thr))

        return lax.fori_loop(0, GROUPS, group_body, carry)

    Tk0 = jnp.full((L,), _NEG_INF, jnp.float32)
    Tv0 = jnp.zeros((L,), jnp.int32)
    thr0 = _NEG_INF

    pltpu.async_copy(src_slice(0), buf0, sem0)

    @pl.loop(0, NCHUNK, step=2, init_carry=(Tk0, Tv0, thr0))
    def chunk_loop(c, carry):
        pltpu.make_async_copy(src_slice(c), buf0, sem0).wait()
        pltpu.async_copy(src_slice(c + 1), buf1, sem1)
        carry = process(buf0, col0 + c * CHUNK, carry)
        pltpu.make_async_copy(src_slice(c + 1), buf1, sem1).wait()

        @pl.when(c + 2 < NCHUNK)
        def _():
            pltpu.async_copy(src_slice(c + 2), buf0, sem0)

        @pl.when(jnp.logical_and(c + 1 == NCHUNK - 1, is_odd))
        def _():
            patch_penalty(buf1)

        return process(buf1, col0 + (c + 1) * CHUNK, carry)

    Tk, Tv, _ = chunk_loop
    kv_v[...] = Tk
    cv_v[...] = Tv
    pltpu.sync_copy(kv_v, keys_out.at[wid])
    pltpu.sync_copy(cv_v, cols_out.at[wid])


def _finalize_body(keys_hbm, cols_hbm, bsum_hbm, t_hbm, bseq_hbm, blogp_hbm,
                   state2_hbm, obseq_hbm, oblogp_hbm, topp_hbm, ostate_hbm,
                   kv, cv, bs, tv, bseq, blogp, obseq, oblogp, topv, qidx2,
                   srows, sem):
    cid = lax.axis_index("c")
    sid = lax.axis_index("s")

    @pl.when(jnp.logical_and(cid == 0, sid == 0))
    def _():
        pltpu.sync_copy(keys_hbm, kv)
        pltpu.sync_copy(cols_hbm, cv)
        pltpu.sync_copy(bsum_hbm, bs)
        pltpu.sync_copy(t_hbm, tv)
        pltpu.sync_copy(bseq_hbm, bseq)
        pltpu.sync_copy(blogp_hbm, blogp)

        lists = []
        for w in range(NW):
            bias = plsc.load_gather(bs, [jnp.full((L,), w >> 1, jnp.int32)])
            kw = kv[w, :] + bias
            pv = jnp.bitwise_or(cv[w, :], np.int32(w << 20))
            lists.append((kw, pv))
        while len(lists) > 1:
            nxt = []
            for i in range(0, len(lists), 2):
                ak, av = lists[i]
                bk, bv = lists[i + 1]
                bdk = lax.rev(bk, (0,))
                bdv = lax.rev(bv, (0,))
                nk = jnp.maximum(ak, bdk)
                nv = jnp.where(ak >= bdk, av, bdv)
                nxt.append(plsc.sort_key_val(nk, nv))
            lists = nxt
        fk, fv = lists[0]
        kd = lax.rev(fk, (0,))
        vd = lax.rev(fv, (0,))

        wv = lax.shift_right_logical(vd, 20)
        q = lax.shift_right_logical(wv, 1)
        cidx = jnp.bitwise_and(vd, np.int32((1 << 20) - 1))

        topv[...] = kd
        pltpu.sync_copy(topv, topp_hbm)

        bq = plsc.load_gather(bs, [q])
        rv = kd - bq
        tvec = tv[...]
        for i in range(SEQ_LEN):
            ivec = jnp.full((L,), i, jnp.int32)
            mlt = ivec < tvec
            meq = ivec == tvec
            row = bseq[i, :]
            gr = plsc.load_gather(bseq, [ivec, q])
            sel = jnp.where(mlt, gr, row)
            obseq[i, :] = jnp.where(meq, cidx, sel)
            frow = blogp[i, :]
            gf = plsc.load_gather(blogp, [ivec, q])
            self_ = jnp.where(mlt, gf, frow)
            oblogp[i, :] = jnp.where(meq, rv, self_)
        pltpu.sync_copy(obseq, obseq_hbm)
        pltpu.sync_copy(oblogp, oblogp_hbm)

        qidx2[pl.ds(0, L)] = q
        qidx2[pl.ds(L, L)] = q + np.int32(L)
        pltpu.async_copy(state2_hbm.at[qidx2], srows, sem).wait()
        pltpu.sync_copy(srows, ostate_hbm)


@functools.lru_cache(maxsize=1)
def _build_calls():
    mesh = plsc.VectorSubcoreMesh(
        core_axis_name="c", subcore_axis_name="s",
        num_cores=NC, num_subcores=NS)

    shard_topk = pl.kernel(
        _shard_topk_body,
        out_type=(
            jax.ShapeDtypeStruct((NW, L), jnp.float32),
            jax.ShapeDtypeStruct((NW, L), jnp.int32),
        ),
        mesh=mesh,
        compiler_params=pltpu.CompilerParams(needs_layout_passes=False),
        scratch_types=[
            pltpu.VMEM((CHUNK,), jnp.float32),
            pltpu.VMEM((CHUNK,), jnp.float32),
            pltpu.VMEM((L,), jnp.float32),
            pltpu.VMEM((L,), jnp.int32),
            pltpu.SemaphoreType.DMA,
            pltpu.SemaphoreType.DMA,
        ],
    )

    finalize = pl.kernel(
        _finalize_body,
        out_type=(
            jax.ShapeDtypeStruct((SEQ_LEN, BEAM), jnp.int32),
            jax.ShapeDtypeStruct((SEQ_LEN, BEAM), jnp.float32),
            jax.ShapeDtypeStruct((BEAM,), jnp.float32),
            jax.ShapeDtypeStruct((2 * BEAM, 1024), jnp.float32),
        ),
        mesh=mesh,
        compiler_params=pltpu.CompilerParams(needs_layout_passes=False),
        scratch_types=[
            pltpu.VMEM((NW, L), jnp.float32),
            pltpu.VMEM((NW, L), jnp.int32),
            pltpu.VMEM((L,), jnp.float32),
            pltpu.VMEM((L,), jnp.int32),
            pltpu.VMEM((SEQ_LEN, BEAM), jnp.int32),
            pltpu.VMEM((SEQ_LEN, BEAM), jnp.float32),
            pltpu.VMEM((SEQ_LEN, BEAM), jnp.int32),
            pltpu.VMEM((SEQ_LEN, BEAM), jnp.float32),
            pltpu.VMEM((L,), jnp.float32),
            pltpu.VMEM((2 * L,), jnp.int32),
            pltpu.VMEM((2 * BEAM, 1024), jnp.float32),
            pltpu.SemaphoreType.DMA,
        ],
    )
    return shard_topk, finalize


def kernel(logprobs, beam_logprobs_sum, beam_seq_logprobs, state, beam_seq,
           t, beam_size):
    del beam_size
    shard_topk, finalize = _build_calls()
    lp_flat = logprobs.reshape(-1)
    keys, cols = shard_topk(lp_flat)
    t_arr = jnp.full((L,), t, dtype=jnp.int32)
    state2 = state.reshape(2 * BEAM, 1024)
    obseq, oblogp, topp, ostate2 = finalize(
        keys, cols, beam_logprobs_sum, t_arr, beam_seq, beam_seq_logprobs,
        state2)
    return obseq, oblogp, topp, ostate2.reshape(2, BEAM, 1024)


# speedup vs baseline: 4.7359x; 4.7359x over previous
import functools

import jax
import jax.numpy as jnp
import numpy as np
from jax import lax
from jax.experimental import pallas as pl
from jax.experimental.pallas import tpu as pltpu
from jax.experimental.pallas import tpu_sc as plsc

BEAM = 16
VOCAB = 1_000_000
SEQ_LEN = 20
RSTRIDE = 1_024_000
RPAD = RSTRIDE - VOCAB
NC, NS, L = 2, 16, 16
NW = NC * NS
SHARD = RSTRIDE // 2
CHUNK = 51_200
NCHUNK = SHARD // CHUNK
G = 25
GROUPS = CHUNK // (G * L)

_NEG_INF = np.float32(-np.inf)


def _shard_topk_body(lp_hbm, keys_out, cols_out, buf0, buf1, kv_v, cv_v,
                     sem0, sem1):
    cid = lax.axis_index("c")
    sid = lax.axis_index("s")
    wid = sid * NC + cid
    base_elem = wid * SHARD
    col0 = (wid % 2) * SHARD
    iota = lax.iota(jnp.int32, L)

    def src_slice(c):
        return lp_hbm.at[pl.ds(base_elem + c * CHUNK, CHUNK)]

    def process(buf, ccol0, carry):
        def group_body(g, carry):
            Tk, Tv, thr = carry
            gbase = g * (G * L)
            vecs = [buf[pl.ds(gbase + j * L, L)] for j in range(G)]
            ms = list(vecs)
            while len(ms) > 1:
                nxt = [jnp.maximum(ms[i], ms[i + 1])
                       for i in range(0, len(ms) - 1, 2)]
                if len(ms) % 2:
                    nxt.append(ms[-1])
                ms = nxt
            gmax = jnp.max(ms[0])

            def rescan(carry):
                Tk, Tv, thr = carry
                for j, X in enumerate(vecs):
                    xm = jnp.max(X)

                    def mrg(a, X=X, j=j):
                        Tk, Tv, _ = a
                        idxv = (ccol0 + gbase + j * L) + iota
                        sk, sv = plsc.sort_key_val(X, idxv, descending=True)
                        nk = jnp.maximum(Tk, sk)
                        nv = jnp.where(Tk >= sk, Tv, sv)
                        Tk2, Tv2 = plsc.sort_key_val(nk, nv)
                        return Tk2, Tv2, jnp.min(Tk2)

                    Tk, Tv, thr = lax.cond(xm > thr, mrg, lambda a: a,
                                           (Tk, Tv, thr))
                return Tk, Tv, thr

            return lax.cond(gmax > thr, rescan, lambda a: a, (Tk, Tv, thr))

        return lax.fori_loop(0, GROUPS, group_body, carry)

    Tk0 = jnp.full((L,), _NEG_INF, jnp.float32)
    Tv0 = jnp.zeros((L,), jnp.int32)
    thr0 = _NEG_INF

    pltpu.async_copy(src_slice(0), buf0, sem0)

    @pl.loop(0, NCHUNK, step=2, init_carry=(Tk0, Tv0, thr0))
    def chunk_loop(c, carry):
        pltpu.make_async_copy(src_slice(c), buf0, sem0).wait()
        pltpu.async_copy(src_slice(c + 1), buf1, sem1)
        carry = process(buf0, col0 + c * CHUNK, carry)
        pltpu.make_async_copy(src_slice(c + 1), buf1, sem1).wait()

        @pl.when(c + 2 < NCHUNK)
        def _():
            pltpu.async_copy(src_slice(c + 2), buf0, sem0)

        return process(buf1, col0 + (c + 1) * CHUNK, carry)

    Tk, Tv, _ = chunk_loop
    kv_v[...] = Tk
    cv_v[...] = Tv
    pltpu.sync_copy(kv_v, keys_out.at[wid])
    pltpu.sync_copy(cv_v, cols_out.at[wid])


def _finalize_body(keys_hbm, cols_hbm, bsum_hbm, t_hbm, bseq_hbm, blogp_hbm,
                   state2_hbm, obseq_hbm, oblogp_hbm, topp_hbm, ostate_hbm,
                   kv, cv, bs, tv, bseq, blogp, obseq, oblogp, topv, qidx2,
                   srows, sem):
    cid = lax.axis_index("c")
    sid = lax.axis_index("s")

    @pl.when(jnp.logical_and(cid == 0, sid == 0))
    def _():
        pltpu.sync_copy(keys_hbm, kv)
        pltpu.sync_copy(cols_hbm, cv)
        pltpu.sync_copy(bsum_hbm, bs)
        pltpu.sync_copy(t_hbm, tv)
        pltpu.sync_copy(bseq_hbm, bseq)
        pltpu.sync_copy(blogp_hbm, blogp)

        lists = []
        for w in range(NW):
            kw = kv[w, :]
            pv = jnp.bitwise_or(cv[w, :], np.int32(w << 20))
            lists.append((kw, pv))
        while len(lists) > 1:
            nxt = []
            for i in range(0, len(lists), 2):
                ak, av = lists[i]
                bk, bv = lists[i + 1]
                bdk = lax.rev(bk, (0,))
                bdv = lax.rev(bv, (0,))
                nk = jnp.maximum(ak, bdk)
                nv = jnp.where(ak >= bdk, av, bdv)
                nxt.append(plsc.sort_key_val(nk, nv))
            lists = nxt
        fk, fv = lists[0]
        kd = lax.rev(fk, (0,))
        vd = lax.rev(fv, (0,))

        wv = lax.shift_right_logical(vd, 20)
        q = lax.shift_right_logical(wv, 1)
        cidx = jnp.bitwise_and(vd, np.int32((1 << 20) - 1))

        topv[...] = kd
        pltpu.sync_copy(topv, topp_hbm)

        bq = plsc.load_gather(bs, [q])
        rv = kd - bq
        tvec = tv[...]
        for i in range(SEQ_LEN):
            ivec = jnp.full((L,), i, jnp.int32)
            mlt = ivec < tvec
            meq = ivec == tvec
            row = bseq[i, :]
            gr = plsc.load_gather(bseq, [ivec, q])
            sel = jnp.where(mlt, gr, row)
            obseq[i, :] = jnp.where(meq, cidx, sel)
            frow = blogp[i, :]
            gf = plsc.load_gather(blogp, [ivec, q])
            self_ = jnp.where(mlt, gf, frow)
            oblogp[i, :] = jnp.where(meq, rv, self_)
        pltpu.sync_copy(obseq, obseq_hbm)
        pltpu.sync_copy(oblogp, oblogp_hbm)

        qidx2[pl.ds(0, L)] = q
        qidx2[pl.ds(L, L)] = q + np.int32(L)
        pltpu.async_copy(state2_hbm.at[qidx2], srows, sem).wait()
        pltpu.sync_copy(srows, ostate_hbm)


@functools.lru_cache(maxsize=1)
def _build_calls():
    mesh = plsc.VectorSubcoreMesh(
        core_axis_name="c", subcore_axis_name="s",
        num_cores=NC, num_subcores=NS)

    shard_topk = pl.kernel(
        _shard_topk_body,
        out_type=(
            jax.ShapeDtypeStruct((NW, L), jnp.float32),
            jax.ShapeDtypeStruct((NW, L), jnp.int32),
        ),
        mesh=mesh,
        compiler_params=pltpu.CompilerParams(needs_layout_passes=False),
        scratch_types=[
            pltpu.VMEM((CHUNK,), jnp.float32),
            pltpu.VMEM((CHUNK,), jnp.float32),
            pltpu.VMEM((L,), jnp.float32),
            pltpu.VMEM((L,), jnp.int32),
            pltpu.SemaphoreType.DMA,
            pltpu.SemaphoreType.DMA,
        ],
    )

    finalize = pl.kernel(
        _finalize_body,
        out_type=(
            jax.ShapeDtypeStruct((SEQ_LEN, BEAM), jnp.int32),
            jax.ShapeDtypeStruct((SEQ_LEN, BEAM), jnp.float32),
            jax.ShapeDtypeStruct((BEAM,), jnp.float32),
            jax.ShapeDtypeStruct((2 * BEAM, 1024), jnp.float32),
        ),
        mesh=mesh,
        compiler_params=pltpu.CompilerParams(needs_layout_passes=False),
        scratch_types=[
            pltpu.VMEM((NW, L), jnp.float32),
            pltpu.VMEM((NW, L), jnp.int32),
            pltpu.VMEM((L,), jnp.float32),
            pltpu.VMEM((L,), jnp.int32),
            pltpu.VMEM((SEQ_LEN, BEAM), jnp.int32),
            pltpu.VMEM((SEQ_LEN, BEAM), jnp.float32),
            pltpu.VMEM((SEQ_LEN, BEAM), jnp.int32),
            pltpu.VMEM((SEQ_LEN, BEAM), jnp.float32),
            pltpu.VMEM((L,), jnp.float32),
            pltpu.VMEM((2 * L,), jnp.int32),
            pltpu.VMEM((2 * BEAM, 1024), jnp.float32),
            pltpu.SemaphoreType.DMA,
        ],
    )
    return shard_topk, finalize


def kernel(logprobs, beam_logprobs_sum, beam_seq_logprobs, state, beam_seq,
           t, beam_size):
    del beam_size
    shard_topk, finalize = _build_calls()
    pen = jnp.where(jnp.arange(VOCAB, dtype=jnp.int32) == VOCAB - 1,
                    np.float32(-1000.0), np.float32(0.0))
    lpb = (logprobs + pen[None, :]) + beam_logprobs_sum[:, None]
    padded = jnp.concatenate(
        [lpb, jnp.full((BEAM, RPAD), _NEG_INF, jnp.float32)], axis=1)
    flat = padded.reshape(-1)
    keys, cols = shard_topk(flat)
    t_arr = jnp.full((L,), t, dtype=jnp.int32)
    state2 = state.reshape(2 * BEAM, 1024)
    obseq, oblogp, topp, ostate2 = finalize(
        keys, cols, beam_logprobs_sum, t_arr, beam_seq, beam_seq_logprobs,
        state2)
    return obseq, oblogp, topp, ostate2.reshape(2, BEAM, 1024)
